# Initial kernel scaffold; baseline (speedup 1.0000x reference)
import jax
import jax.numpy as jnp
from jax.experimental import pallas as pl


def kernel(hidden_states, router_w, shared_gate_w, shared_up_w, shared_down_w, shared_expert_gate_w, eg_w, eg_s, eu_w, eu_s, ed_w, ed_s):
    raise NotImplementedError("write your pallas kernel here")


# speedup vs baseline: 2.3959x; 2.3959x over previous
import functools

import jax
import jax.numpy as jnp
from jax.experimental import pallas as pl
from jax.experimental.pallas import tpu as pltpu

BLK = 128


def _moe_body(x_ref, rw_ref, sg_ref, su_ref, sd_ref, seg_ref,
              egw_ref, euw_ref, edw_ref, egs_ref, eus_ref, eds_ref,
              out_ref, m_ref, gq_ref, uq_ref, dq_ref):
    T, D = x_ref.shape
    DFF = egw_ref.shape[1]
    e = pl.program_id(0)
    t = pl.program_id(1)
    TBLK = T // pl.num_programs(1)
    rows = pl.ds(t * TBLK, TBLK)

    @pl.when(e == 0)
    def _router_and_shared():
        xb = x_ref[rows, :]
        logits = jax.lax.dot_general(
            xb.astype(jnp.bfloat16), rw_ref[...].astype(jnp.bfloat16),
            (((1,), (1,)), ((), ())),
            preferred_element_type=jnp.float32)
        mx = jnp.max(logits, axis=1, keepdims=True)
        ex = jnp.exp(logits - mx)
        p = ex / jnp.sum(ex, axis=1, keepdims=True)
        lane = jax.lax.broadcasted_iota(jnp.int32, p.shape, 1).astype(
            jnp.float32)
        m1 = jnp.max(p, axis=1, keepdims=True)
        i1 = jnp.min(jnp.where(p == m1, lane, jnp.float32(1e9)), axis=1,
                     keepdims=True)
        p2 = jnp.where(lane == i1, jnp.float32(-1e30), p)
        m2 = jnp.max(p2, axis=1, keepdims=True)
        i2 = jnp.min(jnp.where(p2 == m2, lane, jnp.float32(1e9)), axis=1,
                     keepdims=True)
        den = m1 + m2
        m_ref[rows, 0:1] = i1
        m_ref[rows, 1:2] = i2
        m_ref[rows, 2:3] = m1 / den
        m_ref[rows, 3:4] = m2 / den

        xb16 = xb.astype(jnp.bfloat16)
        sgm = jax.lax.dot_general(xb16, sg_ref[...], (((1,), (1,)), ((), ())),
                                  preferred_element_type=jnp.float32)
        sup = jax.lax.dot_general(xb16, su_ref[...], (((1,), (1,)), ((), ())),
                                  preferred_element_type=jnp.float32)
        sh = (sgm * jax.nn.sigmoid(sgm) * sup).astype(jnp.bfloat16)
        shared = jax.lax.dot_general(sh, sd_ref[...], (((1,), (1,)), ((), ())),
                                     preferred_element_type=jnp.float32)
        glog = jax.lax.dot_general(
            xb, seg_ref[...], (((1,), (1,)), ((), ())),
            preferred_element_type=jnp.float32,
            precision=jax.lax.Precision.HIGHEST)
        out_ref[rows, :] = jax.nn.sigmoid(glog) * shared

    @pl.when(t == 0)
    def _dequant():
        for i in range(DFF // BLK):
            for j in range(D // BLK):
                ri = slice(i * BLK, (i + 1) * BLK)
                rj = slice(j * BLK, (j + 1) * BLK)
                gq_ref[ri, rj] = (egw_ref[0, ri, rj]
                                  * egs_ref[e, i, j]).astype(jnp.bfloat16)
                uq_ref[ri, rj] = (euw_ref[0, ri, rj]
                                  * eus_ref[e, i, j]).astype(jnp.bfloat16)
        for i in range(D // BLK):
            for j in range(DFF // BLK):
                ri = slice(i * BLK, (i + 1) * BLK)
                rj = slice(j * BLK, (j + 1) * BLK)
                dq_ref[ri, rj] = (edw_ref[0, ri, rj]
                                  * eds_ref[e, i, j]).astype(jnp.bfloat16)

    xb16 = x_ref[rows, :].astype(jnp.bfloat16)
    gate = jax.lax.dot_general(xb16, gq_ref[...], (((1,), (1,)), ((), ())),
                               preferred_element_type=jnp.float32)
    up = jax.lax.dot_general(xb16, uq_ref[...], (((1,), (1,)), ((), ())),
                             preferred_element_type=jnp.float32)
    h = (gate * jax.nn.sigmoid(gate) * up).astype(jnp.bfloat16)
    y = jax.lax.dot_general(h, dq_ref[...], (((1,), (1,)), ((), ())),
                            preferred_element_type=jnp.float32)
    ef = e.astype(jnp.float32)
    w = (jnp.where(m_ref[rows, 0:1] == ef, m_ref[rows, 2:3], 0.0)
         + jnp.where(m_ref[rows, 1:2] == ef, m_ref[rows, 3:4], 0.0))
    out_ref[rows, :] += y * w


def kernel(hidden_states, router_w, shared_gate_w, shared_up_w, shared_down_w,
           shared_expert_gate_w, eg_w, eg_s, eu_w, eu_s, ed_w, ed_s):
    bsz, seq, D = hidden_states.shape
    T = bsz * seq
    E, DFF, _ = eg_w.shape
    DSH = shared_gate_w.shape[0]
    TB = min(256, T)
    NT = T // TB
    x = hidden_states.reshape(T, D)

    sg16 = shared_gate_w.astype(jnp.bfloat16)
    su16 = shared_up_w.astype(jnp.bfloat16)
    sd16 = shared_down_w.astype(jnp.bfloat16)

    whole = lambda *shape: pl.BlockSpec(shape, lambda e, t: (0,) * len(shape))
    out = pl.pallas_call(
        _moe_body,
        grid=(E, NT),
        in_specs=[
            whole(T, D),
            whole(E, D),
            whole(DSH, D), whole(DSH, D), whole(D, DSH),
            whole(1, D),
            pl.BlockSpec((1, DFF, D), lambda e, t: (e, 0, 0)),
            pl.BlockSpec((1, DFF, D), lambda e, t: (e, 0, 0)),
            pl.BlockSpec((1, D, DFF), lambda e, t: (e, 0, 0)),
            pl.BlockSpec(memory_space=pltpu.SMEM),
            pl.BlockSpec(memory_space=pltpu.SMEM),
            pl.BlockSpec(memory_space=pltpu.SMEM),
        ],
        out_specs=whole(T, D),
        out_shape=jax.ShapeDtypeStruct((T, D), jnp.float32),
        scratch_shapes=[
            pltpu.VMEM((T, 8), jnp.float32),
            pltpu.VMEM((DFF, D), jnp.bfloat16),
            pltpu.VMEM((DFF, D), jnp.bfloat16),
            pltpu.VMEM((D, DFF), jnp.bfloat16),
        ],
    )(x, router_w, sg16, su16, sd16, shared_expert_gate_w,
      eg_w, eu_w, ed_w, eg_s, eu_s, ed_s)
    return out.reshape(bsz, seq, D)
